# Initial kernel scaffold; baseline (speedup 1.0000x reference)
import jax
import jax.numpy as jnp
from jax.experimental import pallas as pl


def kernel(x, edge_index, batch, W1, b1, W2, b2, W_fc, b_fc):
    raise NotImplementedError("write your pallas kernel here")


# speedup vs baseline: 17.5128x; 17.5128x over previous
import functools

import jax
import jax.numpy as jnp
from jax import lax
from jax.experimental import pallas as pl
from jax.experimental.pallas import tpu as pltpu
from jax.experimental.pallas import tpu_sc as plsc

N = 50000
E = 800000
IN_D = 128
HID = 64
HQ = 16
NQ = HID // HQ
QPC = NQ // 2
OUT_D = 10
G = 16

NC = 2
NS = 16
C = 128
CH = 392
PER = CH * C
PAD_E = NS * PER - E
K = 8
ROUNDS = CH // K
DEG_K = 4
DEG_CH = CH // NC
DEG_ROUNDS = DEG_CH // DEG_K
ZROWS = 128
ZCOPIES = 25
ACC_ROWS = NS * ZROWS * ZCOPIES
OUT_ROWS = 50048
OUT_PER_SUB = OUT_ROWS // NS

NB = 2000
NGRID = N // NB


def _fill_rows(ref, nrows, ncols, val):
  def body(i, carry):
    for c0 in range(0, ncols, 16):
      ref[i, pl.ds(c0, 16)] = jnp.full((16,), val, ref.dtype)
    return carry
  lax.fori_loop(0, nrows, body, 0)


def _zero_acc(acc, zbuf, sid):
  def body(k, carry):
    acc_row = (sid * ZCOPIES + k) * ZROWS
    pltpu.sync_copy(zbuf, acc.at[pl.ds(acc_row, ZROWS)])
    return carry
  lax.fori_loop(0, ZCOPIES, body, 0)


def _deg_body(cols_hbm, out_hbm, idxc, buf, acc, ssem):
  cid = lax.axis_index("c")
  sid = lax.axis_index("s")
  _fill_rows(buf, ZROWS, 16, 0.0)
  _zero_acc(acc, buf, sid)
  _fill_rows(buf, ZROWS, 16, 1.0)
  plsc.subcore_barrier()

  def round_body(r, carry):
    j0 = cid * DEG_CH + r * DEG_K
    pltpu.sync_copy(cols_hbm.at[sid, pl.ds(j0, DEG_K)], idxc)
    descs = [pltpu.async_copy(buf, acc.at[idxc.at[b]], ssem, add=True)
             for b in range(DEG_K)]
    for d in descs:
      d.wait()
    return carry
  lax.fori_loop(0, DEG_ROUNDS, round_body, 0)

  plsc.subcore_barrier()
  pltpu.sync_copy(acc.at[pl.ds(sid * OUT_PER_SUB, OUT_PER_SUB)],
                  out_hbm.at[cid, pl.ds(sid * OUT_PER_SUB, OUT_PER_SUB)])


@functools.cache
def _deg_call():
  mesh = plsc.VectorSubcoreMesh(
      core_axis_name="c", subcore_axis_name="s", num_cores=NC, num_subcores=NS)
  return pl.kernel(
      _deg_body,
      out_type=jax.ShapeDtypeStruct((NC, OUT_ROWS, 16), jnp.float32),
      mesh=mesh,
      compiler_params=pltpu.CompilerParams(use_tc_tiling_on_sc=False),
      scratch_types=[
          pltpu.VMEM((DEG_K, C), jnp.int32),
          pltpu.VMEM((ZROWS, 16), jnp.float32),
          pltpu.VMEM_SHARED((ACC_ROWS, 16), jnp.float32),
          pltpu.SemaphoreType.DMA,
      ],
  )


def _agg_body(rows_hbm, cols_hbm, hs_hbm, out_hbm,
              idxr, idxc, vals, zbuf, acc, gsem, ssem):
  cid = lax.axis_index("c")
  sid = lax.axis_index("s")
  _fill_rows(zbuf, ZROWS, HQ, 0.0)

  for q in range(QPC):
    _zero_acc(acc, zbuf, sid)
    plsc.subcore_barrier()

    def round_body(r, carry):
      j0 = r * K
      pltpu.sync_copy(rows_hbm.at[cid * QPC + q, sid, pl.ds(j0, K)], idxr)
      pltpu.sync_copy(cols_hbm.at[sid, pl.ds(j0, K)], idxc)
      gds = [pltpu.async_copy(hs_hbm.at[idxr.at[b]], vals.at[b], gsem)
             for b in range(K)]
      for d in gds:
        d.wait()
      sds = [pltpu.async_copy(vals.at[b], acc.at[idxc.at[b]], ssem, add=True)
             for b in range(K)]
      for d in sds:
        d.wait()
      return carry
    lax.fori_loop(0, ROUNDS, round_body, 0)

    plsc.subcore_barrier()
    pltpu.sync_copy(acc.at[pl.ds(sid * OUT_PER_SUB, OUT_PER_SUB)],
                    out_hbm.at[cid * QPC + q,
                               pl.ds(sid * OUT_PER_SUB, OUT_PER_SUB)])
    plsc.subcore_barrier()


@functools.cache
def _agg_call():
  mesh = plsc.VectorSubcoreMesh(
      core_axis_name="c", subcore_axis_name="s", num_cores=NC, num_subcores=NS)
  return pl.kernel(
      _agg_body,
      out_type=jax.ShapeDtypeStruct((NQ, OUT_ROWS, HQ), jnp.float32),
      mesh=mesh,
      compiler_params=pltpu.CompilerParams(use_tc_tiling_on_sc=False),
      scratch_types=[
          pltpu.VMEM((K, C), jnp.int32),
          pltpu.VMEM((K, C), jnp.int32),
          pltpu.VMEM((K, C, HQ), jnp.float32),
          pltpu.VMEM((ZROWS, HQ), jnp.float32),
          pltpu.VMEM_SHARED((ACC_ROWS, HQ), jnp.float32),
          pltpu.SemaphoreType.DMA,
          pltpu.SemaphoreType.DMA,
      ],
  )


def _dinv_of(deg_ref):
  return lax.rsqrt(deg_ref[0, :, 0:1] + deg_ref[1, :, 0:1] + 1.0)


def _mm1_body(x_ref, w_ref, deg_ref, out_ref):
  dinv = _dinv_of(deg_ref)
  h = jnp.dot(x_ref[...], w_ref[...], preferred_element_type=jnp.float32)
  hs = h * dinv
  for q in range(NQ):
    out_ref[q] = hs[:, q * HQ:(q + 1) * HQ]


_mm1 = pl.pallas_call(
    _mm1_body,
    grid=(NGRID,),
    in_specs=[
        pl.BlockSpec((NB, IN_D), lambda i: (i, 0)),
        pl.BlockSpec((IN_D, HID), lambda i: (0, 0)),
        pl.BlockSpec((NC, NB, 16), lambda i: (0, i, 0)),
    ],
    out_specs=pl.BlockSpec((NQ, NB, HQ), lambda i: (0, i, 0)),
    out_shape=jax.ShapeDtypeStruct((NQ, N, HQ), jnp.float32),
)


def _mid_body(agg_ref, hs_ref, deg_ref, w_ref, b_ref, out_ref):
  dinv = _dinv_of(deg_ref)
  s = jnp.concatenate([agg_ref[q] + hs_ref[q] for q in range(NQ)], axis=1)
  h1 = jnp.maximum(dinv * s + b_ref[...], 0.0)
  hs2 = jnp.dot(h1, w_ref[...], preferred_element_type=jnp.float32) * dinv
  for q in range(NQ):
    out_ref[q] = hs2[:, q * HQ:(q + 1) * HQ]


_mid = pl.pallas_call(
    _mid_body,
    grid=(NGRID,),
    in_specs=[
        pl.BlockSpec((NQ, NB, HQ), lambda i: (0, i, 0)),
        pl.BlockSpec((NQ, NB, HQ), lambda i: (0, i, 0)),
        pl.BlockSpec((NC, NB, 16), lambda i: (0, i, 0)),
        pl.BlockSpec((HID, HID), lambda i: (0, 0)),
        pl.BlockSpec((1, HID), lambda i: (0, 0)),
    ],
    out_specs=pl.BlockSpec((NQ, NB, HQ), lambda i: (0, i, 0)),
    out_shape=jax.ShapeDtypeStruct((NQ, N, HQ), jnp.float32),
)


def _fin_body(agg_ref, hs_ref, deg_ref, b_ref, bat_ref, wfc_ref, bfc_ref,
              out_ref, sc_ref):
  j = pl.program_id(0)
  dinv = _dinv_of(deg_ref)
  s = jnp.concatenate([agg_ref[q] + hs_ref[q] for q in range(NQ)], axis=1)
  h2 = jnp.maximum(dinv * s + b_ref[...], 0.0)
  gids = lax.broadcasted_iota(jnp.int32, (1, G), 1).astype(jnp.float32)
  onehot = (bat_ref[...] == gids).astype(jnp.float32)
  ext = jnp.concatenate([h2, jnp.ones((NB, G), jnp.float32)], axis=1)
  prod = lax.dot_general(onehot, ext, (((0,), (0,)), ((), ())),
                         preferred_element_type=jnp.float32)

  @pl.when(j == 0)
  def _():
    sc_ref[...] = jnp.zeros_like(sc_ref)

  sc_ref[:, :HID + G] += prod

  @pl.when(j == NGRID - 1)
  def _():
    seg = sc_ref[:, :HID]
    cnt = jnp.maximum(sc_ref[:, HID:HID + 1], 1.0)
    out_ref[...] = (jnp.dot(seg / cnt, wfc_ref[...],
                            preferred_element_type=jnp.float32) + bfc_ref[...])


_fin = pl.pallas_call(
    _fin_body,
    grid=(NGRID,),
    in_specs=[
        pl.BlockSpec((NQ, NB, HQ), lambda i: (0, i, 0)),
        pl.BlockSpec((NQ, NB, HQ), lambda i: (0, i, 0)),
        pl.BlockSpec((NC, NB, 16), lambda i: (0, i, 0)),
        pl.BlockSpec((1, HID), lambda i: (0, 0)),
        pl.BlockSpec((NB, 1), lambda i: (i, 0)),
        pl.BlockSpec((HID, OUT_D), lambda i: (0, 0)),
        pl.BlockSpec((1, OUT_D), lambda i: (0, 0)),
    ],
    out_specs=pl.BlockSpec((G, OUT_D), lambda i: (0, 0)),
    out_shape=jax.ShapeDtypeStruct((G, OUT_D), jnp.float32),
    scratch_shapes=[pltpu.VMEM((G, 128), jnp.float32)],
)


def kernel(x, edge_index, batch, W1, b1, W2, b2, W_fc, b_fc):
  row = edge_index[0].astype(jnp.int32)
  col = edge_index[1].astype(jnp.int32)
  rowp = jnp.concatenate([row, jnp.zeros((PAD_E,), jnp.int32)])
  colp = jnp.concatenate([col, jnp.full((PAD_E,), N, jnp.int32)])
  rows3 = rowp.reshape(NS, CH, C)
  rows5 = jnp.stack([rows3 + q * N for q in range(NQ)])
  cols3 = colp.reshape(NS, CH, C)

  deg = _deg_call()(cols3)
  hs1 = _mm1(x, W1, deg)
  agg1 = _agg_call()(rows5, cols3, hs1.reshape(NQ * N, HQ))
  hs2 = _mid(agg1, hs1, deg, W2, b1.reshape(1, HID))
  agg2 = _agg_call()(rows5, cols3, hs2.reshape(NQ * N, HQ))
  bat = batch.astype(jnp.float32).reshape(N, 1)
  out = _fin(agg2, hs2, deg, b2.reshape(1, HID), bat, W_fc,
             b_fc.reshape(1, OUT_D))
  return (out, None)


# speedup vs baseline: 22.3243x; 1.2747x over previous
import functools

import jax
import jax.numpy as jnp
from jax import lax
from jax.experimental import pallas as pl
from jax.experimental.pallas import tpu as pltpu
from jax.experimental.pallas import tpu_sc as plsc

N = 50000
E = 800000
IN_D = 128
HID = 64
HQ = 16
NQ = HID // HQ
QPC = NQ // 2
OUT_D = 10
G = 16

NC = 2
NS = 16
C = 128
CH = 392
PER = CH * C
PAD_E = NS * PER - E
K = 8
ROUNDS = CH // K
DEG_K = 4
DEG_CH = CH // NC
DEG_ROUNDS = DEG_CH // DEG_K
ZROWS = 128
ZCOPIES = 25
ACC_ROWS = NS * ZROWS * ZCOPIES
OUT_ROWS = 50048
OUT_PER_SUB = OUT_ROWS // NS

NB = 2000
NGRID = N // NB


def _fill_rows(ref, nrows, ncols, val):
  def body(i, carry):
    for c0 in range(0, ncols, 16):
      ref[i, pl.ds(c0, 16)] = jnp.full((16,), val, ref.dtype)
    return carry
  lax.fori_loop(0, nrows, body, 0)


def _zero_acc(acc, zbuf, sid):
  def body(k, carry):
    acc_row = (sid * ZCOPIES + k) * ZROWS
    pltpu.sync_copy(zbuf, acc.at[pl.ds(acc_row, ZROWS)])
    return carry
  lax.fori_loop(0, ZCOPIES, body, 0)


def _deg_body(cols_hbm, out_hbm, idxc, buf, acc, ssem):
  cid = lax.axis_index("c")
  sid = lax.axis_index("s")
  _fill_rows(buf, ZROWS, 16, 0.0)
  _zero_acc(acc, buf, sid)
  _fill_rows(buf, ZROWS, 16, 1.0)
  plsc.subcore_barrier()

  def round_body(r, carry):
    j0 = cid * DEG_CH + r * DEG_K
    pltpu.sync_copy(cols_hbm.at[sid, pl.ds(j0, DEG_K)], idxc)
    descs = [pltpu.async_copy(buf, acc.at[idxc.at[b]], ssem, add=True)
             for b in range(DEG_K)]
    for d in descs:
      d.wait()
    return carry
  lax.fori_loop(0, DEG_ROUNDS, round_body, 0)

  plsc.subcore_barrier()
  pltpu.sync_copy(acc.at[pl.ds(sid * OUT_PER_SUB, OUT_PER_SUB)],
                  out_hbm.at[cid, pl.ds(sid * OUT_PER_SUB, OUT_PER_SUB)])


@functools.cache
def _deg_call():
  mesh = plsc.VectorSubcoreMesh(
      core_axis_name="c", subcore_axis_name="s", num_cores=NC, num_subcores=NS)
  return pl.kernel(
      _deg_body,
      out_type=jax.ShapeDtypeStruct((NC, OUT_ROWS, 16), jnp.float32),
      mesh=mesh,
      compiler_params=pltpu.CompilerParams(use_tc_tiling_on_sc=False),
      scratch_types=[
          pltpu.VMEM((DEG_K, C), jnp.int32),
          pltpu.VMEM((ZROWS, 16), jnp.float32),
          pltpu.VMEM_SHARED((ACC_ROWS, 16), jnp.float32),
          pltpu.SemaphoreType.DMA,
      ],
  )


def _agg_body(rows_hbm, cols_hbm, hs_hbm, out_hbm,
              idxr_b, idxc_b, vals_b, zbuf, acc, isem, gsem, ssem):
  cid = lax.axis_index("c")
  sid = lax.axis_index("s")
  _fill_rows(zbuf, ZROWS, HQ, 0.0)

  for q in range(QPC):
    qidx = cid * QPC + q
    _zero_acc(acc, zbuf, sid)
    plsc.subcore_barrier()

    def idx_load(t, slot):
      pltpu.async_copy(rows_hbm.at[qidx, sid, pl.ds(t * K, K)],
                       idxr_b.at[slot], isem)
      pltpu.async_copy(cols_hbm.at[sid, pl.ds(t * K, K)],
                       idxc_b.at[slot], isem)

    def idx_wait():
      pltpu.make_async_copy(rows_hbm.at[qidx, sid, pl.ds(0, K)],
                            idxr_b.at[0], isem).wait()
      pltpu.make_async_copy(cols_hbm.at[sid, pl.ds(0, K)],
                            idxc_b.at[0], isem).wait()

    def gath_start(s3, s2):
      for b in range(K):
        pltpu.async_copy(hs_hbm.at[idxr_b.at[s3, b]], vals_b.at[s2, b], gsem)

    def gath_wait():
      for b in range(K):
        pltpu.make_async_copy(hs_hbm.at[idxr_b.at[0, 0]],
                              vals_b.at[0, b], gsem).wait()

    def scat_start(s3, s2):
      for b in range(K):
        pltpu.async_copy(vals_b.at[s2, b], acc.at[idxc_b.at[s3, b]], ssem,
                         add=True)

    def scat_wait():
      for b in range(K):
        pltpu.make_async_copy(vals_b.at[0, b], acc.at[idxc_b.at[0, b]],
                              ssem).wait()

    idx_load(0, 0)
    idx_load(1, 1)
    idx_wait()
    gath_start(0, 0)
    gath_wait()
    scat_start(0, 0)
    idx_wait()
    gath_start(1, 1)
    idx_load(2, 2)

    def steady(r, carry):
      s3 = lax.rem(r, 3)
      s2 = lax.rem(r, 2)
      n3 = lax.rem(r + 1, 3)
      n2 = lax.rem(r + 1, 2)
      l3 = lax.rem(r + 2, 3)
      gath_wait()
      scat_start(s3, s2)
      idx_wait()
      scat_wait()
      gath_start(n3, n2)
      idx_load(r + 2, l3)
      return carry
    lax.fori_loop(1, ROUNDS - 2, steady, 0)

    r = ROUNDS - 2
    gath_wait()
    scat_start(r % 3, r % 2)
    idx_wait()
    scat_wait()
    gath_start((r + 1) % 3, (r + 1) % 2)
    r = ROUNDS - 1
    gath_wait()
    scat_start(r % 3, r % 2)
    scat_wait()
    scat_wait()

    plsc.subcore_barrier()
    pltpu.sync_copy(acc.at[pl.ds(sid * OUT_PER_SUB, OUT_PER_SUB)],
                    out_hbm.at[cid * QPC + q,
                               pl.ds(sid * OUT_PER_SUB, OUT_PER_SUB)])
    plsc.subcore_barrier()


@functools.cache
def _agg_call():
  mesh = plsc.VectorSubcoreMesh(
      core_axis_name="c", subcore_axis_name="s", num_cores=NC, num_subcores=NS)
  return pl.kernel(
      _agg_body,
      out_type=jax.ShapeDtypeStruct((NQ, OUT_ROWS, HQ), jnp.float32),
      mesh=mesh,
      compiler_params=pltpu.CompilerParams(use_tc_tiling_on_sc=False),
      scratch_types=[
          pltpu.VMEM((3, K, C), jnp.int32),
          pltpu.VMEM((3, K, C), jnp.int32),
          pltpu.VMEM((2, K, C, HQ), jnp.float32),
          pltpu.VMEM((ZROWS, HQ), jnp.float32),
          pltpu.VMEM_SHARED((ACC_ROWS, HQ), jnp.float32),
          pltpu.SemaphoreType.DMA,
          pltpu.SemaphoreType.DMA,
          pltpu.SemaphoreType.DMA,
      ],
  )


def _dinv_of(deg_ref):
  return lax.rsqrt(deg_ref[0, :, 0:1] + deg_ref[1, :, 0:1] + 1.0)


def _mm1_body(x_ref, w_ref, deg_ref, out_ref):
  dinv = _dinv_of(deg_ref)
  h = jnp.dot(x_ref[...], w_ref[...], preferred_element_type=jnp.float32)
  hs = h * dinv
  for q in range(NQ):
    out_ref[q] = hs[:, q * HQ:(q + 1) * HQ]


_mm1 = pl.pallas_call(
    _mm1_body,
    grid=(NGRID,),
    in_specs=[
        pl.BlockSpec((NB, IN_D), lambda i: (i, 0)),
        pl.BlockSpec((IN_D, HID), lambda i: (0, 0)),
        pl.BlockSpec((NC, NB, 16), lambda i: (0, i, 0)),
    ],
    out_specs=pl.BlockSpec((NQ, NB, HQ), lambda i: (0, i, 0)),
    out_shape=jax.ShapeDtypeStruct((NQ, N, HQ), jnp.float32),
)


def _mid_body(agg_ref, hs_ref, deg_ref, w_ref, b_ref, out_ref):
  dinv = _dinv_of(deg_ref)
  s = jnp.concatenate([agg_ref[q] + hs_ref[q] for q in range(NQ)], axis=1)
  h1 = jnp.maximum(dinv * s + b_ref[...], 0.0)
  hs2 = jnp.dot(h1, w_ref[...], preferred_element_type=jnp.float32) * dinv
  for q in range(NQ):
    out_ref[q] = hs2[:, q * HQ:(q + 1) * HQ]


_mid = pl.pallas_call(
    _mid_body,
    grid=(NGRID,),
    in_specs=[
        pl.BlockSpec((NQ, NB, HQ), lambda i: (0, i, 0)),
        pl.BlockSpec((NQ, NB, HQ), lambda i: (0, i, 0)),
        pl.BlockSpec((NC, NB, 16), lambda i: (0, i, 0)),
        pl.BlockSpec((HID, HID), lambda i: (0, 0)),
        pl.BlockSpec((1, HID), lambda i: (0, 0)),
    ],
    out_specs=pl.BlockSpec((NQ, NB, HQ), lambda i: (0, i, 0)),
    out_shape=jax.ShapeDtypeStruct((NQ, N, HQ), jnp.float32),
)


def _fin_body(agg_ref, hs_ref, deg_ref, b_ref, bat_ref, wfc_ref, bfc_ref,
              out_ref, sc_ref):
  j = pl.program_id(0)
  dinv = _dinv_of(deg_ref)
  s = jnp.concatenate([agg_ref[q] + hs_ref[q] for q in range(NQ)], axis=1)
  h2 = jnp.maximum(dinv * s + b_ref[...], 0.0)
  gids = lax.broadcasted_iota(jnp.int32, (1, G), 1).astype(jnp.float32)
  onehot = (bat_ref[...] == gids).astype(jnp.float32)
  ext = jnp.concatenate([h2, jnp.ones((NB, G), jnp.float32)], axis=1)
  prod = lax.dot_general(onehot, ext, (((0,), (0,)), ((), ())),
                         preferred_element_type=jnp.float32)

  @pl.when(j == 0)
  def _():
    sc_ref[...] = jnp.zeros_like(sc_ref)

  sc_ref[:, :HID + G] += prod

  @pl.when(j == NGRID - 1)
  def _():
    seg = sc_ref[:, :HID]
    cnt = jnp.maximum(sc_ref[:, HID:HID + 1], 1.0)
    out_ref[...] = (jnp.dot(seg / cnt, wfc_ref[...],
                            preferred_element_type=jnp.float32) + bfc_ref[...])


_fin = pl.pallas_call(
    _fin_body,
    grid=(NGRID,),
    in_specs=[
        pl.BlockSpec((NQ, NB, HQ), lambda i: (0, i, 0)),
        pl.BlockSpec((NQ, NB, HQ), lambda i: (0, i, 0)),
        pl.BlockSpec((NC, NB, 16), lambda i: (0, i, 0)),
        pl.BlockSpec((1, HID), lambda i: (0, 0)),
        pl.BlockSpec((NB, 1), lambda i: (i, 0)),
        pl.BlockSpec((HID, OUT_D), lambda i: (0, 0)),
        pl.BlockSpec((1, OUT_D), lambda i: (0, 0)),
    ],
    out_specs=pl.BlockSpec((G, OUT_D), lambda i: (0, 0)),
    out_shape=jax.ShapeDtypeStruct((G, OUT_D), jnp.float32),
    scratch_shapes=[pltpu.VMEM((G, 128), jnp.float32)],
)


def kernel(x, edge_index, batch, W1, b1, W2, b2, W_fc, b_fc):
  row = edge_index[0].astype(jnp.int32)
  col = edge_index[1].astype(jnp.int32)
  rowp = jnp.concatenate([row, jnp.zeros((PAD_E,), jnp.int32)])
  colp = jnp.concatenate([col, jnp.full((PAD_E,), N, jnp.int32)])
  rows3 = rowp.reshape(NS, CH, C)
  rows5 = jnp.stack([rows3 + q * N for q in range(NQ)])
  cols3 = colp.reshape(NS, CH, C)

  deg = _deg_call()(cols3)
  hs1 = _mm1(x, W1, deg)
  agg1 = _agg_call()(rows5, cols3, hs1.reshape(NQ * N, HQ))
  hs2 = _mid(agg1, hs1, deg, W2, b1.reshape(1, HID))
  agg2 = _agg_call()(rows5, cols3, hs2.reshape(NQ * N, HQ))
  bat = batch.astype(jnp.float32).reshape(N, 1)
  out = _fin(agg2, hs2, deg, b2.reshape(1, HID), bat, W_fc,
             b_fc.reshape(1, OUT_D))
  return (out, None)


# speedup vs baseline: 31.2545x; 1.4000x over previous
import functools

import jax
import jax.numpy as jnp
from jax import lax
from jax.experimental import pallas as pl
from jax.experimental.pallas import tpu as pltpu
from jax.experimental.pallas import tpu_sc as plsc

N = 50000
NPAD = 51200
E = 800000
IN_D = 128
HID = 64
HQ = 16
NQ = HID // HQ
QPC = NQ // 2
OUT_D = 10
G = 16

NC = 2
NS = 16
C = 128
CH = 392
PER = CH * C
PAD_E = NS * PER - E
K = 8
ROUNDS = CH // K
DEG_K = 4
DEG_CH = CH // NC
DEG_ROUNDS = DEG_CH // DEG_K
ZROWS = 128
ZCOPIES = 25
ACC_ROWS = NS * ZROWS * ZCOPIES
OUT_PER_SUB = ACC_ROWS // NS

NB = 2048
NGRID = NPAD // NB


def _fill_rows(ref, nrows, ncols, val):
  def body(i, carry):
    for c0 in range(0, ncols, 16):
      ref[i, pl.ds(c0, 16)] = jnp.full((16,), val, ref.dtype)
    return carry
  lax.fori_loop(0, nrows, body, 0)


def _zero_acc(acc, zbuf, sid):
  def body(k, carry):
    acc_row = (sid * ZCOPIES + k) * ZROWS
    pltpu.sync_copy(zbuf, acc.at[pl.ds(acc_row, ZROWS)])
    return carry
  lax.fori_loop(0, ZCOPIES, body, 0)


def _deg_body(cols_hbm, out_hbm, idxc, buf, acc, ssem):
  cid = lax.axis_index("c")
  sid = lax.axis_index("s")
  _fill_rows(buf, ZROWS, 16, 0.0)
  _zero_acc(acc, buf, sid)
  _fill_rows(buf, ZROWS, 16, 1.0)
  plsc.subcore_barrier()

  def round_body(r, carry):
    j0 = cid * DEG_CH + r * DEG_K
    pltpu.sync_copy(cols_hbm.at[sid, pl.ds(j0, DEG_K)], idxc)
    descs = [pltpu.async_copy(buf, acc.at[idxc.at[b]], ssem, add=True)
             for b in range(DEG_K)]
    for d in descs:
      d.wait()
    return carry
  lax.fori_loop(0, DEG_ROUNDS, round_body, 0)

  plsc.subcore_barrier()
  pltpu.sync_copy(acc.at[pl.ds(sid * OUT_PER_SUB, OUT_PER_SUB)],
                  out_hbm.at[cid, pl.ds(sid * OUT_PER_SUB, OUT_PER_SUB)])


@functools.cache
def _deg_call():
  mesh = plsc.VectorSubcoreMesh(
      core_axis_name="c", subcore_axis_name="s", num_cores=NC, num_subcores=NS)
  return pl.kernel(
      _deg_body,
      out_type=jax.ShapeDtypeStruct((NC, ACC_ROWS, 16), jnp.float32),
      mesh=mesh,
      compiler_params=pltpu.CompilerParams(use_tc_tiling_on_sc=False),
      scratch_types=[
          pltpu.VMEM((DEG_K, C), jnp.int32),
          pltpu.VMEM((ZROWS, 16), jnp.float32),
          pltpu.VMEM_SHARED((ACC_ROWS, 16), jnp.float32),
          pltpu.SemaphoreType.DMA,
      ],
  )


def _agg_body(rows_hbm, cols_hbm, hs_hbm, out_hbm,
              idxr_b, idxc_b, vals_b, zbuf, acc, isem, gsem, ssem):
  cid = lax.axis_index("c")
  sid = lax.axis_index("s")
  _fill_rows(zbuf, ZROWS, HQ, 0.0)

  for q in range(QPC):
    qidx = cid * QPC + q
    _zero_acc(acc, zbuf, sid)
    plsc.subcore_barrier()

    def idx_load(t, slot):
      pltpu.async_copy(rows_hbm.at[qidx, sid, pl.ds(t * K, K)],
                       idxr_b.at[slot], isem)
      pltpu.async_copy(cols_hbm.at[sid, pl.ds(t * K, K)],
                       idxc_b.at[slot], isem)

    def idx_wait():
      pltpu.make_async_copy(rows_hbm.at[qidx, sid, pl.ds(0, K)],
                            idxr_b.at[0], isem).wait()
      pltpu.make_async_copy(cols_hbm.at[sid, pl.ds(0, K)],
                            idxc_b.at[0], isem).wait()

    def gath_start(s3, s2):
      for b in range(K):
        pltpu.async_copy(hs_hbm.at[idxr_b.at[s3, b]], vals_b.at[s2, b], gsem)

    def gath_wait():
      for b in range(K):
        pltpu.make_async_copy(hs_hbm.at[idxr_b.at[0, 0]],
                              vals_b.at[0, b], gsem).wait()

    def scat_start(s3, s2):
      for b in range(K):
        pltpu.async_copy(vals_b.at[s2, b], acc.at[idxc_b.at[s3, b]], ssem,
                         add=True)

    def scat_wait():
      for b in range(K):
        pltpu.make_async_copy(vals_b.at[0, b], acc.at[idxc_b.at[0, b]],
                              ssem).wait()

    idx_load(0, 0)
    idx_load(1, 1)
    idx_wait()
    gath_start(0, 0)
    gath_wait()
    scat_start(0, 0)
    idx_wait()
    gath_start(1, 1)
    idx_load(2, 2)

    def steady(r, carry):
      s3 = lax.rem(r, 3)
      s2 = lax.rem(r, 2)
      n3 = lax.rem(r + 1, 3)
      n2 = lax.rem(r + 1, 2)
      l3 = lax.rem(r + 2, 3)
      gath_wait()
      scat_start(s3, s2)
      idx_wait()
      scat_wait()
      gath_start(n3, n2)
      idx_load(r + 2, l3)
      return carry
    lax.fori_loop(1, ROUNDS - 2, steady, 0)

    r = ROUNDS - 2
    gath_wait()
    scat_start(r % 3, r % 2)
    idx_wait()
    scat_wait()
    gath_start((r + 1) % 3, (r + 1) % 2)
    r = ROUNDS - 1
    gath_wait()
    scat_start(r % 3, r % 2)
    scat_wait()
    scat_wait()

    plsc.subcore_barrier()
    pltpu.sync_copy(acc.at[pl.ds(sid * OUT_PER_SUB, OUT_PER_SUB)],
                    out_hbm.at[pl.ds(sid * OUT_PER_SUB, OUT_PER_SUB),
                               pl.ds(qidx * HQ, HQ)])
    plsc.subcore_barrier()


@functools.cache
def _agg_call():
  mesh = plsc.VectorSubcoreMesh(
      core_axis_name="c", subcore_axis_name="s", num_cores=NC, num_subcores=NS)
  return pl.kernel(
      _agg_body,
      out_type=jax.ShapeDtypeStruct((NPAD, IN_D), jnp.float32),
      mesh=mesh,
      compiler_params=pltpu.CompilerParams(use_tc_tiling_on_sc=False),
      scratch_types=[
          pltpu.VMEM((3, K, C), jnp.int32),
          pltpu.VMEM((3, K, C), jnp.int32),
          pltpu.VMEM((2, K, C, HQ), jnp.float32),
          pltpu.VMEM((ZROWS, HQ), jnp.float32),
          pltpu.VMEM_SHARED((ACC_ROWS, HQ), jnp.float32),
          pltpu.SemaphoreType.DMA,
          pltpu.SemaphoreType.DMA,
          pltpu.SemaphoreType.DMA,
      ],
  )


def _mm1_body(x_ref, w_ref, deg_ref, out_ref):
  dinv = lax.rsqrt(deg_ref[0, :, 0:1] + deg_ref[1, :, 0:1] + 1.0)
  h = jnp.dot(x_ref[...], w_ref[...], preferred_element_type=jnp.float32)
  out_ref[:, 0:HID] = h * dinv
  out_ref[:, HID:HID + 1] = dinv


_mm1 = pl.pallas_call(
    _mm1_body,
    grid=(NGRID,),
    in_specs=[
        pl.BlockSpec((NB, IN_D), lambda i: (i, 0)),
        pl.BlockSpec((IN_D, HID), lambda i: (0, 0)),
        pl.BlockSpec((NC, NB, 16), lambda i: (0, i, 0)),
    ],
    out_specs=pl.BlockSpec((NB, IN_D), lambda i: (i, 0)),
    out_shape=jax.ShapeDtypeStruct((NPAD, IN_D), jnp.float32),
)


def _mid_body(agg_ref, hs_ref, w_ref, b_ref, out_ref):
  dinv = hs_ref[:, HID:HID + 1]
  s = agg_ref[:, 0:HID] + hs_ref[:, 0:HID]
  h1 = jnp.maximum(dinv * s + b_ref[...], 0.0)
  hs2 = jnp.dot(h1, w_ref[...], preferred_element_type=jnp.float32) * dinv
  out_ref[:, 0:HID] = hs2
  out_ref[:, HID:HID + 1] = dinv


_mid = pl.pallas_call(
    _mid_body,
    grid=(NGRID,),
    in_specs=[
        pl.BlockSpec((NB, IN_D), lambda i: (i, 0)),
        pl.BlockSpec((NB, IN_D), lambda i: (i, 0)),
        pl.BlockSpec((HID, HID), lambda i: (0, 0)),
        pl.BlockSpec((1, HID), lambda i: (0, 0)),
    ],
    out_specs=pl.BlockSpec((NB, IN_D), lambda i: (i, 0)),
    out_shape=jax.ShapeDtypeStruct((NPAD, IN_D), jnp.float32),
)


def _fin_body(agg_ref, hs_ref, b_ref, bat_ref, wfc_ref, bfc_ref,
              out_ref, sc_ref):
  j = pl.program_id(0)
  dinv = hs_ref[:, HID:HID + 1]
  s = agg_ref[:, 0:HID] + hs_ref[:, 0:HID]
  h2 = jnp.maximum(dinv * s + b_ref[...], 0.0)
  rowid = lax.broadcasted_iota(jnp.int32, (NB, 1), 0) + j * NB
  h2 = jnp.where(rowid < N, h2, 0.0)
  gids = lax.broadcasted_iota(jnp.int32, (G, 1), 0).astype(jnp.float32)
  onehot_t = (bat_ref[0] == gids).astype(jnp.float32)
  ext = jnp.concatenate([h2, jnp.ones((NB, G), jnp.float32)], axis=1)
  prod = lax.dot_general(onehot_t, ext, (((1,), (0,)), ((), ())),
                         preferred_element_type=jnp.float32)

  @pl.when(j == 0)
  def _():
    sc_ref[...] = jnp.zeros_like(sc_ref)

  sc_ref[:, :HID + G] += prod

  @pl.when(j == NGRID - 1)
  def _():
    seg = sc_ref[:, :HID]
    cnt = jnp.maximum(sc_ref[:, HID:HID + 1], 1.0)
    out_ref[...] = (jnp.dot(seg / cnt, wfc_ref[...],
                            preferred_element_type=jnp.float32) + bfc_ref[...])


_fin = pl.pallas_call(
    _fin_body,
    grid=(NGRID,),
    in_specs=[
        pl.BlockSpec((NB, IN_D), lambda i: (i, 0)),
        pl.BlockSpec((NB, IN_D), lambda i: (i, 0)),
        pl.BlockSpec((1, HID), lambda i: (0, 0)),
        pl.BlockSpec((1, 1, NB), lambda i: (i, 0, 0)),
        pl.BlockSpec((HID, OUT_D), lambda i: (0, 0)),
        pl.BlockSpec((1, OUT_D), lambda i: (0, 0)),
    ],
    out_specs=pl.BlockSpec((G, OUT_D), lambda i: (0, 0)),
    out_shape=jax.ShapeDtypeStruct((G, OUT_D), jnp.float32),
    scratch_shapes=[pltpu.VMEM((G, 128), jnp.float32)],
)


def kernel(x, edge_index, batch, W1, b1, W2, b2, W_fc, b_fc):
  row = edge_index[0].astype(jnp.int32)
  col = edge_index[1].astype(jnp.int32)
  rowp = jnp.concatenate([row, jnp.zeros((PAD_E,), jnp.int32)])
  colp = jnp.concatenate([col, jnp.full((PAD_E,), N, jnp.int32)])
  rows3 = rowp.reshape(NS, CH, C)
  rows5 = jnp.stack([rows3 * 8 + q for q in range(NQ)])
  cols3 = colp.reshape(NS, CH, C)

  deg = _deg_call()(cols3)
  hs1 = _mm1(x, W1, deg)
  agg1 = _agg_call()(rows5, cols3, hs1.reshape(NPAD * 8, HQ))
  hs2 = _mid(agg1, hs1, W2, b1.reshape(1, HID))
  agg2 = _agg_call()(rows5, cols3, hs2.reshape(NPAD * 8, HQ))
  batf = jnp.concatenate([batch.astype(jnp.float32),
                          jnp.full((NPAD - N,), float(G), jnp.float32)])
  out = _fin(agg2, hs2, b2.reshape(1, HID), batf.reshape(NGRID, 1, NB),
             W_fc, b_fc.reshape(1, OUT_D))
  return (out, None)


# speedup vs baseline: 32.7559x; 1.0480x over previous
import functools

import jax
import jax.numpy as jnp
from jax import lax
from jax.experimental import pallas as pl
from jax.experimental.pallas import tpu as pltpu
from jax.experimental.pallas import tpu_sc as plsc

N = 50000
NPAD = 51200
E = 800000
IN_D = 128
HID = 64
HQ = 16
NQ = HID // HQ
QPC = NQ // 2
OUT_D = 10
G = 16

NC = 2
NS = 16
C = 128
CH = 392
PER = CH * C
PAD_E = NS * PER - E
K = 8
ROUNDS = CH // K
DEG_K = 4
DEG_CH = CH // NC
DEG_ROUNDS = DEG_CH // DEG_K
ZROWS = 128
ZCOPIES = 25
ACC_ROWS = NS * ZROWS * ZCOPIES
OUT_PER_SUB = ACC_ROWS // NS

NB = 2048
NGRID = NPAD // NB


def _fill_rows(ref, nrows, ncols, val):
  def body(i, carry):
    for c0 in range(0, ncols, 16):
      ref[i, pl.ds(c0, 16)] = jnp.full((16,), val, ref.dtype)
    return carry
  lax.fori_loop(0, nrows, body, 0)


def _zero_acc(acc, zbuf, sid):
  def body(k, carry):
    acc_row = (sid * ZCOPIES + k) * ZROWS
    pltpu.sync_copy(zbuf, acc.at[pl.ds(acc_row, ZROWS)])
    return carry
  lax.fori_loop(0, ZCOPIES, body, 0)


def _deg_body(cols_hbm, out_hbm, idxc_b, buf, acc, isem, ssem):
  cid = lax.axis_index("c")
  sid = lax.axis_index("s")
  _fill_rows(buf, ZROWS, 16, 0.0)
  _zero_acc(acc, buf, sid)
  _fill_rows(buf, ZROWS, 16, 1.0)
  plsc.subcore_barrier()

  base = cid * DEG_CH

  def idx_load(t, slot):
    pltpu.async_copy(cols_hbm.at[sid, pl.ds(base + t * DEG_K, DEG_K)],
                     idxc_b.at[slot], isem)

  def idx_wait():
    pltpu.make_async_copy(cols_hbm.at[sid, pl.ds(0, DEG_K)],
                          idxc_b.at[0], isem).wait()

  def scat_start(slot):
    for b in range(DEG_K):
      pltpu.async_copy(buf, acc.at[idxc_b.at[slot, b]], ssem, add=True)

  def scat_wait():
    for b in range(DEG_K):
      pltpu.make_async_copy(buf, acc.at[idxc_b.at[0, b]], ssem).wait()

  idx_load(0, 0)
  idx_load(1, 1)
  idx_wait()
  scat_start(0)
  idx_load(2, 2)

  def steady(r, carry):
    idx_wait()
    scat_start(lax.rem(r, 3))
    scat_wait()
    idx_load(r + 2, lax.rem(r + 2, 3))
    return carry
  lax.fori_loop(1, DEG_ROUNDS - 2, steady, 0)

  r = DEG_ROUNDS - 2
  idx_wait()
  scat_start(r % 3)
  scat_wait()
  r = DEG_ROUNDS - 1
  idx_wait()
  scat_start(r % 3)
  scat_wait()
  scat_wait()

  plsc.subcore_barrier()
  pltpu.sync_copy(acc.at[pl.ds(sid * OUT_PER_SUB, OUT_PER_SUB)],
                  out_hbm.at[pl.ds(sid * OUT_PER_SUB, OUT_PER_SUB),
                             pl.ds(cid * 64, 16)])


@functools.cache
def _deg_call():
  mesh = plsc.VectorSubcoreMesh(
      core_axis_name="c", subcore_axis_name="s", num_cores=NC, num_subcores=NS)
  return pl.kernel(
      _deg_body,
      out_type=jax.ShapeDtypeStruct((NPAD, IN_D), jnp.float32),
      mesh=mesh,
      compiler_params=pltpu.CompilerParams(use_tc_tiling_on_sc=False),
      scratch_types=[
          pltpu.VMEM((3, DEG_K, C), jnp.int32),
          pltpu.VMEM((ZROWS, 16), jnp.float32),
          pltpu.VMEM_SHARED((ACC_ROWS, 16), jnp.float32),
          pltpu.SemaphoreType.DMA,
          pltpu.SemaphoreType.DMA,
      ],
  )


def _agg_body(rows_hbm, cols_hbm, hs_hbm, out_hbm,
              idxr_b, idxc_b, vals_b, zbuf, acc, isem, gsem, ssem):
  cid = lax.axis_index("c")
  sid = lax.axis_index("s")
  _fill_rows(zbuf, ZROWS, HQ, 0.0)

  for q in range(QPC):
    qidx = cid * QPC + q
    _zero_acc(acc, zbuf, sid)
    plsc.subcore_barrier()

    def idx_load(t, slot):
      pltpu.async_copy(rows_hbm.at[qidx, sid, pl.ds(t * K, K)],
                       idxr_b.at[slot], isem)
      pltpu.async_copy(cols_hbm.at[sid, pl.ds(t * K, K)],
                       idxc_b.at[slot], isem)

    def idx_wait():
      pltpu.make_async_copy(rows_hbm.at[qidx, sid, pl.ds(0, K)],
                            idxr_b.at[0], isem).wait()
      pltpu.make_async_copy(cols_hbm.at[sid, pl.ds(0, K)],
                            idxc_b.at[0], isem).wait()

    def gath_start(s3, s2):
      for b in range(K):
        pltpu.async_copy(hs_hbm.at[idxr_b.at[s3, b]], vals_b.at[s2, b], gsem)

    def gath_wait():
      for b in range(K):
        pltpu.make_async_copy(hs_hbm.at[idxr_b.at[0, 0]],
                              vals_b.at[0, b], gsem).wait()

    def scat_start(s3, s2):
      for b in range(K):
        pltpu.async_copy(vals_b.at[s2, b], acc.at[idxc_b.at[s3, b]], ssem,
                         add=True)

    def scat_wait():
      for b in range(K):
        pltpu.make_async_copy(vals_b.at[0, b], acc.at[idxc_b.at[0, b]],
                              ssem).wait()

    idx_load(0, 0)
    idx_load(1, 1)
    idx_wait()
    gath_start(0, 0)
    gath_wait()
    scat_start(0, 0)
    idx_wait()
    gath_start(1, 1)
    idx_load(2, 2)

    def steady(r, carry):
      s3 = lax.rem(r, 3)
      s2 = lax.rem(r, 2)
      n3 = lax.rem(r + 1, 3)
      n2 = lax.rem(r + 1, 2)
      l3 = lax.rem(r + 2, 3)
      gath_wait()
      scat_start(s3, s2)
      idx_wait()
      scat_wait()
      gath_start(n3, n2)
      idx_load(r + 2, l3)
      return carry
    lax.fori_loop(1, ROUNDS - 2, steady, 0)

    r = ROUNDS - 2
    gath_wait()
    scat_start(r % 3, r % 2)
    idx_wait()
    scat_wait()
    gath_start((r + 1) % 3, (r + 1) % 2)
    r = ROUNDS - 1
    gath_wait()
    scat_start(r % 3, r % 2)
    scat_wait()
    scat_wait()

    plsc.subcore_barrier()
    pltpu.sync_copy(acc.at[pl.ds(sid * OUT_PER_SUB, OUT_PER_SUB)],
                    out_hbm.at[pl.ds(sid * OUT_PER_SUB, OUT_PER_SUB),
                               pl.ds(qidx * HQ, HQ)])
    plsc.subcore_barrier()


@functools.cache
def _agg_call():
  mesh = plsc.VectorSubcoreMesh(
      core_axis_name="c", subcore_axis_name="s", num_cores=NC, num_subcores=NS)
  return pl.kernel(
      _agg_body,
      out_type=jax.ShapeDtypeStruct((NPAD, IN_D), jnp.float32),
      mesh=mesh,
      compiler_params=pltpu.CompilerParams(use_tc_tiling_on_sc=False),
      scratch_types=[
          pltpu.VMEM((3, K, C), jnp.int32),
          pltpu.VMEM((3, K, C), jnp.int32),
          pltpu.VMEM((2, K, C, HQ), jnp.float32),
          pltpu.VMEM((ZROWS, HQ), jnp.float32),
          pltpu.VMEM_SHARED((ACC_ROWS, HQ), jnp.float32),
          pltpu.SemaphoreType.DMA,
          pltpu.SemaphoreType.DMA,
          pltpu.SemaphoreType.DMA,
      ],
  )


def _mm1_body(x_ref, w_ref, deg_ref, out_ref):
  dinv = lax.rsqrt(deg_ref[:, 0:1] + deg_ref[:, 64:65] + 1.0)
  h = jnp.dot(x_ref[...], w_ref[...], preferred_element_type=jnp.float32)
  out_ref[:, 0:HID] = h * dinv
  out_ref[:, HID:HID + 1] = dinv


_mm1 = pl.pallas_call(
    _mm1_body,
    grid=(NGRID,),
    in_specs=[
        pl.BlockSpec((NB, IN_D), lambda i: (i, 0)),
        pl.BlockSpec((IN_D, HID), lambda i: (0, 0)),
        pl.BlockSpec((NB, IN_D), lambda i: (i, 0)),
    ],
    out_specs=pl.BlockSpec((NB, IN_D), lambda i: (i, 0)),
    out_shape=jax.ShapeDtypeStruct((NPAD, IN_D), jnp.float32),
)


def _mid_body(agg_ref, hs_ref, w_ref, b_ref, out_ref):
  dinv = hs_ref[:, HID:HID + 1]
  s = agg_ref[:, 0:HID] + hs_ref[:, 0:HID]
  h1 = jnp.maximum(dinv * s + b_ref[...], 0.0)
  hs2 = jnp.dot(h1, w_ref[...], preferred_element_type=jnp.float32) * dinv
  out_ref[:, 0:HID] = hs2
  out_ref[:, HID:HID + 1] = dinv


_mid = pl.pallas_call(
    _mid_body,
    grid=(NGRID,),
    in_specs=[
        pl.BlockSpec((NB, IN_D), lambda i: (i, 0)),
        pl.BlockSpec((NB, IN_D), lambda i: (i, 0)),
        pl.BlockSpec((HID, HID), lambda i: (0, 0)),
        pl.BlockSpec((1, HID), lambda i: (0, 0)),
    ],
    out_specs=pl.BlockSpec((NB, IN_D), lambda i: (i, 0)),
    out_shape=jax.ShapeDtypeStruct((NPAD, IN_D), jnp.float32),
)


def _fin_body(agg_ref, hs_ref, b_ref, bat_ref, wfc_ref, bfc_ref,
              out_ref, sc_ref):
  j = pl.program_id(0)
  dinv = hs_ref[:, HID:HID + 1]
  s = agg_ref[:, 0:HID] + hs_ref[:, 0:HID]
  h2 = jnp.maximum(dinv * s + b_ref[...], 0.0)
  rowid = lax.broadcasted_iota(jnp.int32, (NB, 1), 0) + j * NB
  h2 = jnp.where(rowid < N, h2, 0.0)
  gids = lax.broadcasted_iota(jnp.int32, (G, 1), 0).astype(jnp.float32)
  onehot_t = (bat_ref[0] == gids).astype(jnp.float32)
  ext = jnp.concatenate([h2, jnp.ones((NB, G), jnp.float32)], axis=1)
  prod = lax.dot_general(onehot_t, ext, (((1,), (0,)), ((), ())),
                         preferred_element_type=jnp.float32)

  @pl.when(j == 0)
  def _():
    sc_ref[...] = jnp.zeros_like(sc_ref)

  sc_ref[:, :HID + G] += prod

  @pl.when(j == NGRID - 1)
  def _():
    seg = sc_ref[:, :HID]
    cnt = jnp.maximum(sc_ref[:, HID:HID + 1], 1.0)
    out_ref[...] = (jnp.dot(seg / cnt, wfc_ref[...],
                            preferred_element_type=jnp.float32) + bfc_ref[...])


_fin = pl.pallas_call(
    _fin_body,
    grid=(NGRID,),
    in_specs=[
        pl.BlockSpec((NB, IN_D), lambda i: (i, 0)),
        pl.BlockSpec((NB, IN_D), lambda i: (i, 0)),
        pl.BlockSpec((1, HID), lambda i: (0, 0)),
        pl.BlockSpec((1, 1, NB), lambda i: (i, 0, 0)),
        pl.BlockSpec((HID, OUT_D), lambda i: (0, 0)),
        pl.BlockSpec((1, OUT_D), lambda i: (0, 0)),
    ],
    out_specs=pl.BlockSpec((G, OUT_D), lambda i: (0, 0)),
    out_shape=jax.ShapeDtypeStruct((G, OUT_D), jnp.float32),
    scratch_shapes=[pltpu.VMEM((G, 128), jnp.float32)],
)


def kernel(x, edge_index, batch, W1, b1, W2, b2, W_fc, b_fc):
  row = edge_index[0].astype(jnp.int32)
  col = edge_index[1].astype(jnp.int32)
  rowp = jnp.concatenate([row, jnp.zeros((PAD_E,), jnp.int32)])
  colp = jnp.concatenate([col, jnp.full((PAD_E,), N, jnp.int32)])
  rows3 = rowp.reshape(NS, CH, C)
  rows5 = jnp.stack([rows3 * 8 + q for q in range(NQ)])
  cols3 = colp.reshape(NS, CH, C)

  deg = _deg_call()(cols3)
  hs1 = _mm1(x, W1, deg)
  agg1 = _agg_call()(rows5, cols3, hs1.reshape(NPAD * 8, HQ))
  hs2 = _mid(agg1, hs1, W2, b1.reshape(1, HID))
  agg2 = _agg_call()(rows5, cols3, hs2.reshape(NPAD * 8, HQ))
  batf = jnp.concatenate([batch.astype(jnp.float32),
                          jnp.full((NPAD - N,), float(G), jnp.float32)])
  out = _fin(agg2, hs2, b2.reshape(1, HID), batf.reshape(NGRID, 1, NB),
             W_fc, b_fc.reshape(1, OUT_D))
  return (out, None)
